# baseline (speedup 1.0000x reference)
import jax
import jax.numpy as jnp
from jax.experimental import pallas as pl
from jax.experimental.pallas import tpu as pltpu

S_MAX = 10


def _speaker_gru_kernel(x_ref, sp_ref, w_ref, b_ref, out_ref, mem_ref):
    T, d, blk = x_ref.shape

    w = w_ref[...]
    b = b_ref[...]

    mem_ref[...] = jnp.zeros_like(mem_ref)

    bf = jnp.bfloat16

    def step(t, h):
        xt = x_ref[t]
        h32 = h.astype(jnp.float32)
        cat = jnp.concatenate([xt, h], axis=0)
        g = jnp.dot(w, cat, preferred_element_type=jnp.float32) + b
        rz = jax.nn.sigmoid(g[:2 * d])
        r = rz[:d]
        z = rz[d:2 * d]
        n = jnp.tanh(g[2 * d:3 * d] + r * g[3 * d:])
        h_new = n + z * (h32 - n)
        out_ref[t] = h_new
        h_new_bf = h_new.astype(bf)

        sp_t = sp_ref[pl.ds(t, 1), :].astype(bf)
        tn = jnp.minimum(t + 1, T - 1)
        sp_n = sp_ref[pl.ds(tn, 1), :].astype(bf)

        gathered = mem_ref[0:d]
        for s in range(1, S_MAX):
            gathered = jnp.where(sp_n == jnp.asarray(s, bf),
                                 mem_ref[s * d:(s + 1) * d], gathered)
        h_next = jnp.where(sp_n == sp_t, h_new_bf, gathered)

        for s in range(S_MAX):
            sl = slice(s * d, (s + 1) * d)
            mem_ref[sl] = jnp.where(sp_t == jnp.asarray(s, bf),
                                    h_new_bf, mem_ref[sl])

        return h_next

    jax.lax.fori_loop(0, T, step, jnp.zeros((d, blk), bf))


def kernel(x_in, speakers, W_ih, W_hh, b_ih, b_hh):
    B, T, d_in = x_in.shape
    d = W_hh.shape[1]
    b_blk = 1024

    bf = jnp.bfloat16
    sp_t = jnp.clip(speakers, 0, S_MAX - 1).astype(jnp.int32).T
    x_t = jnp.transpose(x_in, (1, 2, 0)).astype(bf)

    zz = jnp.zeros((d, d), W_ih.dtype)
    w_big = jnp.concatenate([
        jnp.concatenate([W_ih[:d], W_hh[:d]], axis=1),
        jnp.concatenate([W_ih[d:2 * d], W_hh[d:2 * d]], axis=1),
        jnp.concatenate([W_ih[2 * d:], zz], axis=1),
        jnp.concatenate([zz, W_hh[2 * d:]], axis=1),
    ], axis=0).astype(bf)
    b_big = jnp.concatenate([
        b_ih[:d] + b_hh[:d],
        b_ih[d:2 * d] + b_hh[d:2 * d],
        b_ih[2 * d:],
        b_hh[2 * d:],
    ]).reshape(4 * d, 1)

    grid = (B // b_blk,)
    out_t = pl.pallas_call(
        _speaker_gru_kernel,
        grid=grid,
        in_specs=[
            pl.BlockSpec((T, d_in, b_blk), lambda i: (0, 0, i)),
            pl.BlockSpec((T, b_blk), lambda i: (0, i)),
            pl.BlockSpec((4 * d, 2 * d), lambda i: (0, 0)),
            pl.BlockSpec((4 * d, 1), lambda i: (0, 0)),
        ],
        out_specs=pl.BlockSpec((T, d, b_blk), lambda i: (0, 0, i)),
        out_shape=jax.ShapeDtypeStruct((T, d, B), x_in.dtype),
        scratch_shapes=[pltpu.VMEM((S_MAX * d, b_blk), jnp.bfloat16)],
        compiler_params=pltpu.CompilerParams(
            dimension_semantics=("arbitrary",),
        ),
    )(x_t, sp_t, w_big, b_big)
    return jnp.transpose(out_t, (2, 0, 1))


# speedup vs baseline: 1.2795x; 1.2795x over previous
import jax
import jax.numpy as jnp
from jax.experimental import pallas as pl
from jax.experimental.pallas import tpu as pltpu

S_MAX = 10


def _speaker_gru_kernel(x_ref, sp_ref, w_ref, b_ref, out_ref, mem_ref):
    T, d, blk = x_ref.shape

    w = w_ref[...]
    b = b_ref[...]

    mem_ref[...] = jnp.zeros_like(mem_ref)

    def step(t, h):
        xt = x_ref[t]
        cat = jnp.concatenate([xt, h], axis=0)
        g = jnp.dot(w, cat, preferred_element_type=jnp.float32) + b
        rz = jax.nn.sigmoid(g[:2 * d])
        r = rz[:d]
        z = rz[d:2 * d]
        n = jnp.tanh(g[2 * d:3 * d] + r * g[3 * d:])
        h_new = n + z * (h - n)
        out_ref[t] = h_new

        sp_t = sp_ref[pl.ds(t, 1), :]
        tn = jnp.minimum(t + 1, T - 1)
        sp_n = sp_ref[pl.ds(tn, 1), :]

        gathered = mem_ref[0:d]
        for s in range(1, S_MAX):
            gathered = jnp.where(sp_n == s, mem_ref[s * d:(s + 1) * d],
                                 gathered)
        h_next = jnp.where(sp_n == sp_t, h_new, gathered)

        for s in range(S_MAX):
            sl = slice(s * d, (s + 1) * d)
            mem_ref[sl] = jnp.where(sp_t == s, h_new, mem_ref[sl])

        return h_next

    jax.lax.fori_loop(0, T, step, jnp.zeros((d, blk), jnp.float32), unroll=2)


def kernel(x_in, speakers, W_ih, W_hh, b_ih, b_hh):
    B, T, d_in = x_in.shape
    d = W_hh.shape[1]
    b_blk = 1024

    sp_t = jnp.clip(speakers, 0, S_MAX - 1).astype(jnp.int32).T
    x_t = jnp.transpose(x_in, (1, 2, 0))

    zz = jnp.zeros((d, d), W_ih.dtype)
    w_big = jnp.concatenate([
        jnp.concatenate([W_ih[:d], W_hh[:d]], axis=1),
        jnp.concatenate([W_ih[d:2 * d], W_hh[d:2 * d]], axis=1),
        jnp.concatenate([W_ih[2 * d:], zz], axis=1),
        jnp.concatenate([zz, W_hh[2 * d:]], axis=1),
    ], axis=0)
    b_big = jnp.concatenate([
        b_ih[:d] + b_hh[:d],
        b_ih[d:2 * d] + b_hh[d:2 * d],
        b_ih[2 * d:],
        b_hh[2 * d:],
    ]).reshape(4 * d, 1)

    grid = (B // b_blk,)
    out_t = pl.pallas_call(
        _speaker_gru_kernel,
        grid=grid,
        in_specs=[
            pl.BlockSpec((T, d_in, b_blk), lambda i: (0, 0, i)),
            pl.BlockSpec((T, b_blk), lambda i: (0, i)),
            pl.BlockSpec((4 * d, 2 * d), lambda i: (0, 0)),
            pl.BlockSpec((4 * d, 1), lambda i: (0, 0)),
        ],
        out_specs=pl.BlockSpec((T, d, b_blk), lambda i: (0, 0, i)),
        out_shape=jax.ShapeDtypeStruct((T, d, B), x_in.dtype),
        scratch_shapes=[pltpu.VMEM((S_MAX * d, b_blk), jnp.float32)],
        compiler_params=pltpu.CompilerParams(
            dimension_semantics=("arbitrary",),
        ),
    )(x_t, sp_t, w_big, b_big)
    return jnp.transpose(out_t, (2, 0, 1))


# speedup vs baseline: 1.3020x; 1.0176x over previous
import jax
import jax.numpy as jnp
from jax.experimental import pallas as pl
from jax.experimental.pallas import tpu as pltpu

S_MAX = 10


def _speaker_gru_kernel(x_ref, sp_ref, w_ref, b_ref, out_ref, mem_ref):
    T, d, blk = x_ref.shape

    w = w_ref[...]
    b = b_ref[...]

    mem_ref[...] = jnp.zeros_like(mem_ref)

    def step(t, h):
        xt = x_ref[t]
        cat = jnp.concatenate([xt, h], axis=0)
        g = jnp.dot(w, cat, preferred_element_type=jnp.float32) + b
        rz = jax.nn.sigmoid(g[:2 * d])
        r = rz[:d]
        z = rz[d:2 * d]
        n = jnp.tanh(g[2 * d:3 * d] + r * g[3 * d:])
        h_new = n + z * (h - n)
        out_ref[t] = h_new

        sp_t = sp_ref[pl.ds(t, 1), :]
        tn = jnp.minimum(t + 1, T - 1)
        sp_n = sp_ref[pl.ds(tn, 1), :]

        gathered = mem_ref[0:d]
        for s in range(1, S_MAX):
            gathered = jnp.where(sp_n == s, mem_ref[s * d:(s + 1) * d],
                                 gathered)
        h_next = jnp.where(sp_n == sp_t, h_new, gathered)

        for s in range(S_MAX):
            sl = slice(s * d, (s + 1) * d)
            mem_ref[sl] = jnp.where(sp_t == s, h_new, mem_ref[sl])

        return h_next

    jax.lax.fori_loop(0, T, step, jnp.zeros((d, blk), jnp.float32), unroll=5)


def kernel(x_in, speakers, W_ih, W_hh, b_ih, b_hh):
    B, T, d_in = x_in.shape
    d = W_hh.shape[1]
    b_blk = 1024

    sp_t = jnp.clip(speakers, 0, S_MAX - 1).astype(jnp.int32).T
    x_t = jnp.transpose(x_in, (1, 2, 0))

    zz = jnp.zeros((d, d), W_ih.dtype)
    w_big = jnp.concatenate([
        jnp.concatenate([W_ih[:d], W_hh[:d]], axis=1),
        jnp.concatenate([W_ih[d:2 * d], W_hh[d:2 * d]], axis=1),
        jnp.concatenate([W_ih[2 * d:], zz], axis=1),
        jnp.concatenate([zz, W_hh[2 * d:]], axis=1),
    ], axis=0)
    b_big = jnp.concatenate([
        b_ih[:d] + b_hh[:d],
        b_ih[d:2 * d] + b_hh[d:2 * d],
        b_ih[2 * d:],
        b_hh[2 * d:],
    ]).reshape(4 * d, 1)

    grid = (B // b_blk,)
    out_t = pl.pallas_call(
        _speaker_gru_kernel,
        grid=grid,
        in_specs=[
            pl.BlockSpec((T, d_in, b_blk), lambda i: (0, 0, i)),
            pl.BlockSpec((T, b_blk), lambda i: (0, i)),
            pl.BlockSpec((4 * d, 2 * d), lambda i: (0, 0)),
            pl.BlockSpec((4 * d, 1), lambda i: (0, 0)),
        ],
        out_specs=pl.BlockSpec((T, d, b_blk), lambda i: (0, 0, i)),
        out_shape=jax.ShapeDtypeStruct((T, d, B), x_in.dtype),
        scratch_shapes=[pltpu.VMEM((S_MAX * d, b_blk), jnp.float32)],
        compiler_params=pltpu.CompilerParams(
            dimension_semantics=("arbitrary",),
        ),
    )(x_t, sp_t, w_big, b_big)
    return jnp.transpose(out_t, (2, 0, 1))


# speedup vs baseline: 1.3039x; 1.0015x over previous
import jax
import jax.numpy as jnp
from jax.experimental import pallas as pl
from jax.experimental.pallas import tpu as pltpu

S_MAX = 10


def _speaker_gru_kernel(x_ref, sp_ref, w_ref, b_ref, out_ref, mem_ref):
    T, d, blk = x_ref.shape

    w = w_ref[...]
    b = b_ref[...]

    mem_ref[...] = jnp.zeros_like(mem_ref)

    def step(t, h):
        xt = x_ref[t]
        cat = jnp.concatenate([xt, h], axis=0)
        g = jnp.dot(w, cat, preferred_element_type=jnp.float32) + b
        rz = jax.nn.sigmoid(g[:2 * d])
        r = rz[:d]
        z = rz[d:2 * d]
        n = jnp.tanh(g[2 * d:3 * d] + r * g[3 * d:])
        h_new = n + z * (h - n)
        out_ref[t] = h_new

        sp_t = sp_ref[pl.ds(t, 1), :]
        tn = jnp.minimum(t + 1, T - 1)
        sp_n = sp_ref[pl.ds(tn, 1), :]

        gathered = mem_ref[0:d]
        for s in range(1, S_MAX):
            gathered = jnp.where(sp_n == s, mem_ref[s * d:(s + 1) * d],
                                 gathered)
        h_next = jnp.where(sp_n == sp_t, h_new, gathered)

        for s in range(S_MAX):
            sl = slice(s * d, (s + 1) * d)
            mem_ref[sl] = jnp.where(sp_t == s, h_new, mem_ref[sl])

        return h_next

    jax.lax.fori_loop(0, T, step, jnp.zeros((d, blk), jnp.float32), unroll=10)


def kernel(x_in, speakers, W_ih, W_hh, b_ih, b_hh):
    B, T, d_in = x_in.shape
    d = W_hh.shape[1]
    b_blk = 1024

    sp_t = jnp.clip(speakers, 0, S_MAX - 1).astype(jnp.int32).T
    x_t = jnp.transpose(x_in, (1, 2, 0))

    zz = jnp.zeros((d, d), W_ih.dtype)
    w_big = jnp.concatenate([
        jnp.concatenate([W_ih[:d], W_hh[:d]], axis=1),
        jnp.concatenate([W_ih[d:2 * d], W_hh[d:2 * d]], axis=1),
        jnp.concatenate([W_ih[2 * d:], zz], axis=1),
        jnp.concatenate([zz, W_hh[2 * d:]], axis=1),
    ], axis=0)
    b_big = jnp.concatenate([
        b_ih[:d] + b_hh[:d],
        b_ih[d:2 * d] + b_hh[d:2 * d],
        b_ih[2 * d:],
        b_hh[2 * d:],
    ]).reshape(4 * d, 1)

    grid = (B // b_blk,)
    out_t = pl.pallas_call(
        _speaker_gru_kernel,
        grid=grid,
        in_specs=[
            pl.BlockSpec((T, d_in, b_blk), lambda i: (0, 0, i)),
            pl.BlockSpec((T, b_blk), lambda i: (0, i)),
            pl.BlockSpec((4 * d, 2 * d), lambda i: (0, 0)),
            pl.BlockSpec((4 * d, 1), lambda i: (0, 0)),
        ],
        out_specs=pl.BlockSpec((T, d, b_blk), lambda i: (0, 0, i)),
        out_shape=jax.ShapeDtypeStruct((T, d, B), x_in.dtype),
        scratch_shapes=[pltpu.VMEM((S_MAX * d, b_blk), jnp.float32)],
        compiler_params=pltpu.CompilerParams(
            dimension_semantics=("arbitrary",),
        ),
    )(x_t, sp_t, w_big, b_big)
    return jnp.transpose(out_t, (2, 0, 1))
